# Initial kernel scaffold; baseline (speedup 1.0000x reference)
import jax
import jax.numpy as jnp
from jax.experimental import pallas as pl


def kernel(x, edge_index, edge_weight, anchor_idxs, W_hidden, b_hidden, W_conv1, W_conv2, W_fc, b_fc, W_class, b_class):
    raise NotImplementedError("write your pallas kernel here")


# speedup vs baseline: 4.7058x; 4.7058x over previous
import functools
import math

import jax
import jax.numpy as jnp
from jax import lax
from jax.experimental import pallas as pl
from jax.experimental.pallas import tpu as pltpu
from jax.experimental.pallas import tpu_sc as plsc

ALPHA = 0.1
THETA = 0.5
EPS = 1e-12

NC = 2
NS = 16
NW = NC * NS
LANES = 16


def _l2n(v):
    n = jnp.sqrt(jnp.sum(v * v, axis=-1, keepdims=True))
    return v / jnp.maximum(n, EPS)


@functools.lru_cache(maxsize=None)
def _make_msgpass(N, D, E, K, PH):
    EW = E // NW
    CH = EW // K
    CP = CH // PH
    assert EW * NW == E and CH * K == EW and CP * PH == CH
    assert K % 8 == 0 and K <= 128
    RPT = ((N + NS - 1) // NS + 7) // 8 * 8
    RLAST = N - RPT * (NS - 1)
    assert RPT % 8 == 0 and 0 < RLAST <= RPT and RLAST % 8 == 0
    mesh = plsc.VectorSubcoreMesh(core_axis_name="c", subcore_axis_name="s")

    @functools.partial(
        pl.kernel,
        out_type=jax.ShapeDtypeStruct((NC, N, D), jnp.float32),
        mesh=mesh,
        scratch_types=[
            pltpu.VMEM_SHARED((N, D), jnp.float32),
            pltpu.VMEM((CP, K), jnp.int32),
            pltpu.VMEM((CP, K), jnp.int32),
            pltpu.VMEM((CP * K,), jnp.float32),
            pltpu.VMEM((K, D), jnp.float32),
            pltpu.SemaphoreType.DMA,
        ],
        compiler_params=pltpu.CompilerParams(needs_layout_passes=False),
    )
    def msgpass(h_hbm, src_hbm, dst_hbm, w_hbm, z_hbm, out_hbm,
                acc, src_all, dst_all, w_all, rows, sem):
        c = lax.axis_index("c")
        s = lax.axis_index("s")
        wid = s * NC + c
        @pl.when(s < NS - 1)
        def _():
            pltpu.sync_copy(z_hbm.at[pl.ds(s * RPT, RPT)],
                            acc.at[pl.ds(s * RPT, RPT)])

        @pl.when(s == NS - 1)
        def _():
            pltpu.sync_copy(z_hbm.at[pl.ds((NS - 1) * RPT, RLAST)],
                            acc.at[pl.ds((NS - 1) * RPT, RLAST)])
        plsc.subcore_barrier()

        def stage(p, carry0):
            pltpu.sync_copy(src_hbm.at[wid, p], src_all)
            pltpu.sync_copy(dst_hbm.at[wid, p], dst_all)
            pltpu.sync_copy(w_hbm.at[wid, p], w_all)

            def chunk(i, carry):
                pltpu.async_copy(h_hbm.at[src_all.at[i]], rows, sem).wait()

                def escale(e, c2):
                    wv = plsc.load_gather(
                        w_all, [jnp.full((LANES,), i * K + e, jnp.int32)])
                    for j in range(D // LANES):
                        sl = pl.ds(j * LANES, LANES)
                        rows[e, sl] = rows[e, sl] * wv
                    return c2

                lax.fori_loop(0, K, escale, 0)
                pltpu.sync_copy(rows, acc.at[dst_all.at[i]], add=True)
                return carry

            lax.fori_loop(0, CP, chunk, 0)
            return carry0

        lax.fori_loop(0, PH, stage, 0)
        plsc.subcore_barrier()

        @pl.when(s < NS - 1)
        def _():
            pltpu.sync_copy(acc.at[pl.ds(s * RPT, RPT)],
                            out_hbm.at[c, pl.ds(s * RPT, RPT)])

        @pl.when(s == NS - 1)
        def _():
            pltpu.sync_copy(acc.at[pl.ds((NS - 1) * RPT, RLAST)],
                            out_hbm.at[c, pl.ds((NS - 1) * RPT, RLAST)])

    return msgpass


@functools.lru_cache(maxsize=None)
def _make_anchor_gather(N, D, A):
    AW = A // NW
    assert AW * NW == A and AW % 8 == 0
    mesh = plsc.VectorSubcoreMesh(core_axis_name="c", subcore_axis_name="s")

    @functools.partial(
        pl.kernel,
        out_type=jax.ShapeDtypeStruct((A, D), jnp.float32),
        mesh=mesh,
        scratch_types=[
            pltpu.VMEM((AW,), jnp.int32),
            pltpu.VMEM((AW, D), jnp.float32),
            pltpu.SemaphoreType.DMA,
        ],
    )
    def gather(h_hbm, idx_hbm, out_hbm, idx_v, rows, sem):
        c = lax.axis_index("c")
        s = lax.axis_index("s")
        base = (s * NC + c) * AW
        pltpu.sync_copy(idx_hbm.at[pl.ds(base, AW)], idx_v)
        pltpu.async_copy(h_hbm.at[idx_v], rows, sem).wait()
        pltpu.sync_copy(rows, out_hbm.at[pl.ds(base, AW)])

    return gather


@functools.lru_cache(maxsize=None)
def _make_dense0(N, D, BN):
    def body(x_ref, w_ref, b_ref, o_ref):
        h = jnp.dot(x_ref[...], w_ref[...], preferred_element_type=jnp.float32)
        o_ref[...] = jnp.maximum(h + b_ref[...], 0.0)

    return pl.pallas_call(
        body,
        grid=(N // BN,),
        in_specs=[
            pl.BlockSpec((BN, D), lambda i: (i, 0)),
            pl.BlockSpec((D, D), lambda i: (0, 0)),
            pl.BlockSpec((1, D), lambda i: (0, 0)),
        ],
        out_specs=pl.BlockSpec((BN, D), lambda i: (i, 0)),
        out_shape=jax.ShapeDtypeStruct((N, D), jnp.float32),
    )


@functools.lru_cache(maxsize=None)
def _make_combine(N, D, BN, beta, with_fc):
    def body(p_ref, x0_ref, w_ref, *rest):
        agg = p_ref[0] + p_ref[1]
        hh = (1.0 - ALPHA) * agg + ALPHA * x0_ref[...]
        hw = jnp.dot(hh, w_ref[...], preferred_element_type=jnp.float32)
        h = jnp.maximum((1.0 - beta) * hh + beta * hw, 0.0)
        if with_fc:
            wfc_ref, bfc_ref, o_ref = rest
            o_ref[...] = (
                jnp.dot(h, wfc_ref[...], preferred_element_type=jnp.float32)
                + bfc_ref[...])
        else:
            (o_ref,) = rest
            o_ref[...] = h

    in_specs = [
        pl.BlockSpec((NC, BN, D), lambda i: (0, i, 0)),
        pl.BlockSpec((BN, D), lambda i: (i, 0)),
        pl.BlockSpec((D, D), lambda i: (0, 0)),
    ]
    if with_fc:
        in_specs += [
            pl.BlockSpec((D, D), lambda i: (0, 0)),
            pl.BlockSpec((1, D), lambda i: (0, 0)),
        ]
    return pl.pallas_call(
        body,
        grid=(N // BN,),
        in_specs=in_specs,
        out_specs=pl.BlockSpec((BN, D), lambda i: (i, 0)),
        out_shape=jax.ShapeDtypeStruct((N, D), jnp.float32),
    )


@functools.lru_cache(maxsize=None)
def _make_head(N, D, A, C, BN):
    def body(h_ref, anc_ref, wc_ref, bc_ref, sims_ref, log_ref):
        hn = _l2n(h_ref[...])
        an = _l2n(anc_ref[...])
        s = jnp.dot(hn, an.T, preferred_element_type=jnp.float32)
        sims_ref[...] = s
        out = _l2n(s)
        log_ref[...] = (
            jnp.dot(out, wc_ref[...], preferred_element_type=jnp.float32)
            + bc_ref[...])

    return pl.pallas_call(
        body,
        grid=(N // BN,),
        in_specs=[
            pl.BlockSpec((BN, D), lambda i: (i, 0)),
            pl.BlockSpec((A, D), lambda i: (0, 0)),
            pl.BlockSpec((A, C), lambda i: (0, 0)),
            pl.BlockSpec((1, C), lambda i: (0, 0)),
        ],
        out_specs=[
            pl.BlockSpec((BN, A), lambda i: (i, 0)),
            pl.BlockSpec((BN, C), lambda i: (i, 0)),
        ],
        out_shape=[
            jax.ShapeDtypeStruct((N, A), jnp.float32),
            jax.ShapeDtypeStruct((N, C), jnp.float32),
        ],
    )


def kernel(x, edge_index, edge_weight, anchor_idxs, W_hidden, b_hidden,
           W_conv1, W_conv2, W_fc, b_fc, W_class, b_class):
    N, D = x.shape
    E = edge_weight.shape[0]
    A = anchor_idxs.shape[0]
    C = W_class.shape[1]
    BN = 1000
    K = 80
    PH = 5
    EW = E // NW
    CH = EW // K
    CP = CH // PH

    msgpass = _make_msgpass(N, D, E, K, PH)
    anchor_gather = _make_anchor_gather(N, D, A)
    dense0 = _make_dense0(N, D, BN)
    beta1 = math.log(THETA / 1.0 + 1.0)
    beta2 = math.log(THETA / 2.0 + 1.0)
    comb1 = _make_combine(N, D, BN, beta1, False)
    comb2 = _make_combine(N, D, BN, beta2, True)
    head = _make_head(N, D, A, C, BN)

    src_r = edge_index[0].reshape(NW, PH, CP, K)
    dst_r = edge_index[1].reshape(NW, PH, CP, K)
    w_r = edge_weight.reshape(NW, PH, CP * K)
    zeros = jnp.zeros((N, D), jnp.float32)

    h0 = dense0(x, W_hidden, b_hidden.reshape(1, D))
    p1 = msgpass(h0, src_r, dst_r, w_r, zeros)
    h1 = comb1(p1, h0, W_conv1)
    p2 = msgpass(h1, src_r, dst_r, w_r, zeros)
    hfc = comb2(p2, h0, W_conv2, W_fc, b_fc.reshape(1, D))
    anchors = anchor_gather(hfc, anchor_idxs)
    sims, logits = head(hfc, anchors, W_class, b_class.reshape(1, C))
    return (logits, sims)


# speedup vs baseline: 9.2537x; 1.9665x over previous
import functools
import math

import jax
import jax.numpy as jnp
from jax import lax
from jax.experimental import pallas as pl
from jax.experimental.pallas import tpu as pltpu
from jax.experimental.pallas import tpu_sc as plsc

ALPHA = 0.1
THETA = 0.5
EPS = 1e-12

NC = 2
NS = 16
NW = NC * NS
LANES = 16


def _l2n(v):
    n = jnp.sqrt(jnp.sum(v * v, axis=-1, keepdims=True))
    return v / jnp.maximum(n, EPS)


@functools.lru_cache(maxsize=None)
def _make_msgpass(N, D, E, K, PH, NB):
    EW = E // NW
    CH = EW // K
    CP = CH // PH
    assert EW * NW == E and CH * K == EW and CP * PH == CH
    assert K % 8 == 0 and K <= 128 and CP % NB == 0
    RPT = ((N + NS - 1) // NS + 7) // 8 * 8
    RLAST = N - RPT * (NS - 1)
    assert RPT % 8 == 0 and 0 < RLAST <= RPT and RLAST % 8 == 0
    mesh = plsc.VectorSubcoreMesh(core_axis_name="c", subcore_axis_name="s")

    @functools.partial(
        pl.kernel,
        out_type=jax.ShapeDtypeStruct((NC, N, D), jnp.float32),
        mesh=mesh,
        scratch_types=[
            pltpu.VMEM_SHARED((N, D), jnp.float32),
            pltpu.VMEM((CP, K), jnp.int32),
            pltpu.VMEM((CP, K), jnp.int32),
            pltpu.VMEM((CP * K,), jnp.float32),
        ] + [pltpu.VMEM((K, D), jnp.float32) for _ in range(NB)]
          + [pltpu.SemaphoreType.DMA for _ in range(2 * NB)],
        compiler_params=pltpu.CompilerParams(needs_layout_passes=False),
    )
    def msgpass(h_hbm, src_hbm, dst_hbm, w_hbm, z_hbm, out_hbm,
                acc, src_all, dst_all, w_all, *bufs):
        rows = bufs[:NB]
        gsem = bufs[NB:2 * NB]
        ssem = bufs[2 * NB:3 * NB]
        c = lax.axis_index("c")
        s = lax.axis_index("s")
        wid = s * NC + c
        @pl.when(s < NS - 1)
        def _():
            pltpu.sync_copy(z_hbm.at[pl.ds(s * RPT, RPT)],
                            acc.at[pl.ds(s * RPT, RPT)])

        @pl.when(s == NS - 1)
        def _():
            pltpu.sync_copy(z_hbm.at[pl.ds((NS - 1) * RPT, RLAST)],
                            acc.at[pl.ds((NS - 1) * RPT, RLAST)])
        plsc.subcore_barrier()

        def gissue(i, b):
            pltpu.async_copy(h_hbm.at[src_all.at[i]], rows[b], gsem[b])

        def gwait(i, b):
            pltpu.make_async_copy(h_hbm.at[src_all.at[i]], rows[b],
                                  gsem[b]).wait()

        def sissue(i, b):
            pltpu.async_copy(rows[b], acc.at[dst_all.at[i]], ssem[b],
                             add=True)

        def swait(b):
            pltpu.make_async_copy(rows[b], acc.at[dst_all.at[0]],
                                  ssem[b]).wait()

        def stage(p, carry0):
            pltpu.sync_copy(src_hbm.at[wid, p], src_all)
            pltpu.sync_copy(dst_hbm.at[wid, p], dst_all)
            pltpu.sync_copy(w_hbm.at[wid, p], w_all)
            for b in range(NB - 1):
                gissue(b, b)

            def group(g, carry):
                for b in range(NB):
                    i = g * NB + b
                    gwait(i, b)

                    def escale(e, c2):
                        wv = plsc.load_gather(
                            w_all, [jnp.full((LANES,), i * K + e, jnp.int32)])
                        for j in range(D // LANES):
                            sl = pl.ds(j * LANES, LANES)
                            rows[b][e, sl] = rows[b][e, sl] * wv
                        return c2

                    lax.fori_loop(0, K, escale, 0, unroll=4)
                    sissue(i, b)
                    j = i + NB - 1
                    bj = (b + NB - 1) % NB

                    @pl.when(jnp.logical_and(i > 0, j < CP))
                    def _():
                        swait(bj)

                    @pl.when(j < CP)
                    def _():
                        gissue(j, bj)
                return carry

            lax.fori_loop(0, CP // NB, group, 0)
            for b in range(NB):
                swait(b)
            return carry0

        lax.fori_loop(0, PH, stage, 0)
        plsc.subcore_barrier()

        @pl.when(s < NS - 1)
        def _():
            pltpu.sync_copy(acc.at[pl.ds(s * RPT, RPT)],
                            out_hbm.at[c, pl.ds(s * RPT, RPT)])

        @pl.when(s == NS - 1)
        def _():
            pltpu.sync_copy(acc.at[pl.ds((NS - 1) * RPT, RLAST)],
                            out_hbm.at[c, pl.ds((NS - 1) * RPT, RLAST)])

    return msgpass


@functools.lru_cache(maxsize=None)
def _make_anchor_gather(N, D, A):
    AW = A // NW
    assert AW * NW == A and AW % 8 == 0
    mesh = plsc.VectorSubcoreMesh(core_axis_name="c", subcore_axis_name="s")

    @functools.partial(
        pl.kernel,
        out_type=jax.ShapeDtypeStruct((A, D), jnp.float32),
        mesh=mesh,
        scratch_types=[
            pltpu.VMEM((AW,), jnp.int32),
            pltpu.VMEM((AW, D), jnp.float32),
            pltpu.SemaphoreType.DMA,
        ],
    )
    def gather(h_hbm, idx_hbm, out_hbm, idx_v, rows, sem):
        c = lax.axis_index("c")
        s = lax.axis_index("s")
        base = (s * NC + c) * AW
        pltpu.sync_copy(idx_hbm.at[pl.ds(base, AW)], idx_v)
        pltpu.async_copy(h_hbm.at[idx_v], rows, sem).wait()
        pltpu.sync_copy(rows, out_hbm.at[pl.ds(base, AW)])

    return gather


@functools.lru_cache(maxsize=None)
def _make_dense0(N, D, BN):
    def body(x_ref, w_ref, b_ref, o_ref):
        h = jnp.dot(x_ref[...], w_ref[...], preferred_element_type=jnp.float32)
        o_ref[...] = jnp.maximum(h + b_ref[...], 0.0)

    return pl.pallas_call(
        body,
        grid=(N // BN,),
        in_specs=[
            pl.BlockSpec((BN, D), lambda i: (i, 0)),
            pl.BlockSpec((D, D), lambda i: (0, 0)),
            pl.BlockSpec((1, D), lambda i: (0, 0)),
        ],
        out_specs=pl.BlockSpec((BN, D), lambda i: (i, 0)),
        out_shape=jax.ShapeDtypeStruct((N, D), jnp.float32),
    )


@functools.lru_cache(maxsize=None)
def _make_combine(N, D, BN, beta, with_fc):
    def body(p_ref, x0_ref, w_ref, *rest):
        agg = p_ref[0] + p_ref[1]
        hh = (1.0 - ALPHA) * agg + ALPHA * x0_ref[...]
        hw = jnp.dot(hh, w_ref[...], preferred_element_type=jnp.float32)
        h = jnp.maximum((1.0 - beta) * hh + beta * hw, 0.0)
        if with_fc:
            wfc_ref, bfc_ref, o_ref = rest
            o_ref[...] = (
                jnp.dot(h, wfc_ref[...], preferred_element_type=jnp.float32)
                + bfc_ref[...])
        else:
            (o_ref,) = rest
            o_ref[...] = h

    in_specs = [
        pl.BlockSpec((NC, BN, D), lambda i: (0, i, 0)),
        pl.BlockSpec((BN, D), lambda i: (i, 0)),
        pl.BlockSpec((D, D), lambda i: (0, 0)),
    ]
    if with_fc:
        in_specs += [
            pl.BlockSpec((D, D), lambda i: (0, 0)),
            pl.BlockSpec((1, D), lambda i: (0, 0)),
        ]
    return pl.pallas_call(
        body,
        grid=(N // BN,),
        in_specs=in_specs,
        out_specs=pl.BlockSpec((BN, D), lambda i: (i, 0)),
        out_shape=jax.ShapeDtypeStruct((N, D), jnp.float32),
    )


@functools.lru_cache(maxsize=None)
def _make_head(N, D, A, C, BN):
    def body(h_ref, anc_ref, wc_ref, bc_ref, sims_ref, log_ref):
        hn = _l2n(h_ref[...])
        an = _l2n(anc_ref[...])
        s = jnp.dot(hn, an.T, preferred_element_type=jnp.float32)
        sims_ref[...] = s
        out = _l2n(s)
        log_ref[...] = (
            jnp.dot(out, wc_ref[...], preferred_element_type=jnp.float32)
            + bc_ref[...])

    return pl.pallas_call(
        body,
        grid=(N // BN,),
        in_specs=[
            pl.BlockSpec((BN, D), lambda i: (i, 0)),
            pl.BlockSpec((A, D), lambda i: (0, 0)),
            pl.BlockSpec((A, C), lambda i: (0, 0)),
            pl.BlockSpec((1, C), lambda i: (0, 0)),
        ],
        out_specs=[
            pl.BlockSpec((BN, A), lambda i: (i, 0)),
            pl.BlockSpec((BN, C), lambda i: (i, 0)),
        ],
        out_shape=[
            jax.ShapeDtypeStruct((N, A), jnp.float32),
            jax.ShapeDtypeStruct((N, C), jnp.float32),
        ],
    )


def kernel(x, edge_index, edge_weight, anchor_idxs, W_hidden, b_hidden,
           W_conv1, W_conv2, W_fc, b_fc, W_class, b_class):
    N, D = x.shape
    E = edge_weight.shape[0]
    A = anchor_idxs.shape[0]
    C = W_class.shape[1]
    BN = 1000
    K = 40
    PH = 5
    NB = 5
    EW = E // NW
    CH = EW // K
    CP = CH // PH

    msgpass = _make_msgpass(N, D, E, K, PH, NB)
    anchor_gather = _make_anchor_gather(N, D, A)
    dense0 = _make_dense0(N, D, BN)
    beta1 = math.log(THETA / 1.0 + 1.0)
    beta2 = math.log(THETA / 2.0 + 1.0)
    comb1 = _make_combine(N, D, BN, beta1, False)
    comb2 = _make_combine(N, D, BN, beta2, True)
    head = _make_head(N, D, A, C, BN)

    src_r = edge_index[0].reshape(NW, PH, CP, K)
    dst_r = edge_index[1].reshape(NW, PH, CP, K)
    w_r = edge_weight.reshape(NW, PH, CP * K)
    zeros = jnp.zeros((N, D), jnp.float32)

    h0 = dense0(x, W_hidden, b_hidden.reshape(1, D))
    p1 = msgpass(h0, src_r, dst_r, w_r, zeros)
    h1 = comb1(p1, h0, W_conv1)
    p2 = msgpass(h1, src_r, dst_r, w_r, zeros)
    hfc = comb2(p2, h0, W_conv2, W_fc, b_fc.reshape(1, D))
    anchors = anchor_gather(hfc, anchor_idxs)
    sims, logits = head(hfc, anchors, W_class, b_class.reshape(1, C))
    return (logits, sims)


# speedup vs baseline: 9.2882x; 1.0037x over previous
import functools
import math

import jax
import jax.numpy as jnp
from jax import lax
from jax.experimental import pallas as pl
from jax.experimental.pallas import tpu as pltpu
from jax.experimental.pallas import tpu_sc as plsc

ALPHA = 0.1
THETA = 0.5
EPS = 1e-12

NC = 2
NS = 16
NW = NC * NS
LANES = 16


def _l2n(v):
    n = jnp.sqrt(jnp.sum(v * v, axis=-1, keepdims=True))
    return v / jnp.maximum(n, EPS)


@functools.lru_cache(maxsize=None)
def _make_msgpass(N, D, E, K, PH, NB):
    EW = E // NW
    CH = EW // K
    CP = CH // PH
    assert EW * NW == E and CH * K == EW and CP * PH == CH
    assert K % 8 == 0 and K <= 128 and CP % NB == 0
    RPT = ((N + NS - 1) // NS + 7) // 8 * 8
    RLAST = N - RPT * (NS - 1)
    assert RPT % 8 == 0 and 0 < RLAST <= RPT and RLAST % 8 == 0
    mesh = plsc.VectorSubcoreMesh(core_axis_name="c", subcore_axis_name="s")

    @functools.partial(
        pl.kernel,
        out_type=jax.ShapeDtypeStruct((NC, N, D), jnp.float32),
        mesh=mesh,
        scratch_types=[
            pltpu.VMEM_SHARED((N, D), jnp.float32),
            pltpu.VMEM((CP, K), jnp.int32),
            pltpu.VMEM((CP, K), jnp.int32),
            pltpu.VMEM((CP * K,), jnp.float32),
        ] + [pltpu.VMEM((K, D), jnp.float32) for _ in range(NB)]
          + [pltpu.SemaphoreType.DMA for _ in range(2 * NB)],
        compiler_params=pltpu.CompilerParams(needs_layout_passes=False),
    )
    def msgpass(h_hbm, src_hbm, dst_hbm, w_hbm, z_hbm, out_hbm,
                acc, src_all, dst_all, w_all, *bufs):
        rows = bufs[:NB]
        gsem = bufs[NB:2 * NB]
        ssem = bufs[2 * NB:3 * NB]
        c = lax.axis_index("c")
        s = lax.axis_index("s")
        wid = s * NC + c
        @pl.when(s < NS - 1)
        def _():
            pltpu.sync_copy(z_hbm.at[pl.ds(s * RPT, RPT)],
                            acc.at[pl.ds(s * RPT, RPT)])

        @pl.when(s == NS - 1)
        def _():
            pltpu.sync_copy(z_hbm.at[pl.ds((NS - 1) * RPT, RLAST)],
                            acc.at[pl.ds((NS - 1) * RPT, RLAST)])
        plsc.subcore_barrier()

        def gissue(i, b):
            pltpu.async_copy(h_hbm.at[src_all.at[i]], rows[b], gsem[b])

        def gwait(i, b):
            pltpu.make_async_copy(h_hbm.at[src_all.at[i]], rows[b],
                                  gsem[b]).wait()

        def sissue(i, b):
            pltpu.async_copy(rows[b], acc.at[dst_all.at[i]], ssem[b],
                             add=True)

        def swait(b):
            pltpu.make_async_copy(rows[b], acc.at[dst_all.at[0]],
                                  ssem[b]).wait()

        def stage(p, carry0):
            pltpu.sync_copy(src_hbm.at[wid, p], src_all)
            pltpu.sync_copy(dst_hbm.at[wid, p], dst_all)
            pltpu.sync_copy(w_hbm.at[wid, p], w_all)
            for b in range(NB - 1):
                gissue(b, b)

            def group(g, carry):
                for b in range(NB):
                    i = g * NB + b
                    gwait(i, b)

                    @plsc.parallel_loop(0, K, unroll=8)
                    def _(e):
                        wv = plsc.load_gather(
                            w_all, [jnp.full((LANES,), i * K + e, jnp.int32)])
                        for j in range(D // LANES):
                            sl = pl.ds(j * LANES, LANES)
                            rows[b][e, sl] = rows[b][e, sl] * wv
                    sissue(i, b)
                    j = i + NB - 1
                    bj = (b + NB - 1) % NB

                    @pl.when(jnp.logical_and(i > 0, j < CP))
                    def _():
                        swait(bj)

                    @pl.when(j < CP)
                    def _():
                        gissue(j, bj)
                return carry

            lax.fori_loop(0, CP // NB, group, 0)
            for b in range(NB):
                swait(b)
            return carry0

        lax.fori_loop(0, PH, stage, 0)
        plsc.subcore_barrier()

        @pl.when(s < NS - 1)
        def _():
            pltpu.sync_copy(acc.at[pl.ds(s * RPT, RPT)],
                            out_hbm.at[c, pl.ds(s * RPT, RPT)])

        @pl.when(s == NS - 1)
        def _():
            pltpu.sync_copy(acc.at[pl.ds((NS - 1) * RPT, RLAST)],
                            out_hbm.at[c, pl.ds((NS - 1) * RPT, RLAST)])

    return msgpass


@functools.lru_cache(maxsize=None)
def _make_dense0(N, D, BN):
    def body(x_ref, w_ref, b_ref, o_ref):
        h = jnp.dot(x_ref[...], w_ref[...], preferred_element_type=jnp.float32)
        o_ref[...] = jnp.maximum(h + b_ref[...], 0.0)

    return pl.pallas_call(
        body,
        grid=(N // BN,),
        in_specs=[
            pl.BlockSpec((BN, D), lambda i: (i, 0)),
            pl.BlockSpec((D, D), lambda i: (0, 0)),
            pl.BlockSpec((1, D), lambda i: (0, 0)),
        ],
        out_specs=pl.BlockSpec((BN, D), lambda i: (i, 0)),
        out_shape=jax.ShapeDtypeStruct((N, D), jnp.float32),
    )


@functools.lru_cache(maxsize=None)
def _make_combine(N, D, BN, beta, with_fc, A=None):
    def body(p_ref, x0_ref, w_ref, *rest):
        agg = p_ref[0] + p_ref[1]
        hh = (1.0 - ALPHA) * agg + ALPHA * x0_ref[...]
        hw = jnp.dot(hh, w_ref[...], preferred_element_type=jnp.float32)
        h = jnp.maximum((1.0 - beta) * hh + beta * hw, 0.0)
        if with_fc:
            wfc_ref, bfc_ref, aidx_ref, o_ref, anc_ref = rest
            hfc = (
                jnp.dot(h, wfc_ref[...], preferred_element_type=jnp.float32)
                + bfc_ref[...])
            o_ref[...] = hfc
            i = pl.program_id(0)
            row_ids = (jax.lax.broadcasted_iota(jnp.int32, (BN, A), 0)
                       + i * BN)
            onehot = (row_ids == aidx_ref[...]).astype(jnp.float32)
            part = jax.lax.dot_general(
                onehot, hfc, (((0,), (0,)), ((), ())),
                preferred_element_type=jnp.float32)

            @pl.when(i == 0)
            def _():
                anc_ref[...] = jnp.zeros_like(anc_ref)

            anc_ref[...] += part
        else:
            (o_ref,) = rest
            o_ref[...] = h

    in_specs = [
        pl.BlockSpec((NC, BN, D), lambda i: (0, i, 0)),
        pl.BlockSpec((BN, D), lambda i: (i, 0)),
        pl.BlockSpec((D, D), lambda i: (0, 0)),
    ]
    if with_fc:
        in_specs += [
            pl.BlockSpec((D, D), lambda i: (0, 0)),
            pl.BlockSpec((1, D), lambda i: (0, 0)),
            pl.BlockSpec((1, A), lambda i: (0, 0)),
        ]
        out_specs = [
            pl.BlockSpec((BN, D), lambda i: (i, 0)),
            pl.BlockSpec((A, D), lambda i: (0, 0)),
        ]
        out_shape = [
            jax.ShapeDtypeStruct((N, D), jnp.float32),
            jax.ShapeDtypeStruct((A, D), jnp.float32),
        ]
    else:
        out_specs = pl.BlockSpec((BN, D), lambda i: (i, 0))
        out_shape = jax.ShapeDtypeStruct((N, D), jnp.float32)
    return pl.pallas_call(
        body,
        grid=(N // BN,),
        in_specs=in_specs,
        out_specs=out_specs,
        out_shape=out_shape,
    )


@functools.lru_cache(maxsize=None)
def _make_head(N, D, A, C, BN):
    def body(h_ref, anc_ref, wc_ref, bc_ref, sims_ref, log_ref):
        hn = _l2n(h_ref[...])
        an = _l2n(anc_ref[...])
        s = jnp.dot(hn, an.T, preferred_element_type=jnp.float32)
        sims_ref[...] = s
        out = _l2n(s)
        log_ref[...] = (
            jnp.dot(out, wc_ref[...], preferred_element_type=jnp.float32)
            + bc_ref[...])

    return pl.pallas_call(
        body,
        grid=(N // BN,),
        in_specs=[
            pl.BlockSpec((BN, D), lambda i: (i, 0)),
            pl.BlockSpec((A, D), lambda i: (0, 0)),
            pl.BlockSpec((A, C), lambda i: (0, 0)),
            pl.BlockSpec((1, C), lambda i: (0, 0)),
        ],
        out_specs=[
            pl.BlockSpec((BN, A), lambda i: (i, 0)),
            pl.BlockSpec((BN, C), lambda i: (i, 0)),
        ],
        out_shape=[
            jax.ShapeDtypeStruct((N, A), jnp.float32),
            jax.ShapeDtypeStruct((N, C), jnp.float32),
        ],
    )


def kernel(x, edge_index, edge_weight, anchor_idxs, W_hidden, b_hidden,
           W_conv1, W_conv2, W_fc, b_fc, W_class, b_class):
    N, D = x.shape
    E = edge_weight.shape[0]
    A = anchor_idxs.shape[0]
    C = W_class.shape[1]
    BN = 1000
    K = 40
    PH = 5
    NB = 5
    EW = E // NW
    CH = EW // K
    CP = CH // PH

    msgpass = _make_msgpass(N, D, E, K, PH, NB)
    dense0 = _make_dense0(N, D, BN)
    beta1 = math.log(THETA / 1.0 + 1.0)
    beta2 = math.log(THETA / 2.0 + 1.0)
    comb1 = _make_combine(N, D, BN, beta1, False)
    comb2 = _make_combine(N, D, BN, beta2, True, A)
    head = _make_head(N, D, A, C, BN)

    src_r = edge_index[0].reshape(NW, PH, CP, K)
    dst_r = edge_index[1].reshape(NW, PH, CP, K)
    w_r = edge_weight.reshape(NW, PH, CP * K)
    zeros = jnp.zeros((N, D), jnp.float32)

    h0 = dense0(x, W_hidden, b_hidden.reshape(1, D))
    p1 = msgpass(h0, src_r, dst_r, w_r, zeros)
    h1 = comb1(p1, h0, W_conv1)
    p2 = msgpass(h1, src_r, dst_r, w_r, zeros)
    hfc, anchors = comb2(p2, h0, W_conv2, W_fc, b_fc.reshape(1, D),
                         anchor_idxs.reshape(1, A))
    sims, logits = head(hfc, anchors, W_class, b_class.reshape(1, C))
    return (logits, sims)


# speedup vs baseline: 9.6905x; 1.0433x over previous
import functools
import math

import jax
import jax.numpy as jnp
from jax import lax
from jax.experimental import pallas as pl
from jax.experimental.pallas import tpu as pltpu
from jax.experimental.pallas import tpu_sc as plsc

ALPHA = 0.1
THETA = 0.5
EPS = 1e-12

NC = 2
NS = 16
NW = NC * NS
LANES = 16


def _l2n(v):
    n = jnp.sqrt(jnp.sum(v * v, axis=-1, keepdims=True))
    return v / jnp.maximum(n, EPS)


@functools.lru_cache(maxsize=None)
def _make_msgpass(N, D, E, K, PH, NB):
    EW = E // NW
    CH = EW // K
    CP = CH // PH
    NG = CP // NB
    TAIL = CP - NG * NB
    assert EW * NW == E and CH * K == EW and CP * PH == CH
    assert K % 8 == 0 and K <= 128
    RPT = ((N + NS - 1) // NS + 7) // 8 * 8
    RLAST = N - RPT * (NS - 1)
    assert RPT % 8 == 0 and 0 < RLAST <= RPT and RLAST % 8 == 0
    mesh = plsc.VectorSubcoreMesh(core_axis_name="c", subcore_axis_name="s")

    @functools.partial(
        pl.kernel,
        out_type=jax.ShapeDtypeStruct((NC, N, D), jnp.float32),
        mesh=mesh,
        scratch_types=[
            pltpu.VMEM_SHARED((N, D), jnp.float32),
            pltpu.VMEM((CP, K), jnp.int32),
            pltpu.VMEM((CP, K), jnp.int32),
            pltpu.VMEM((CP * K,), jnp.float32),
        ] + [pltpu.VMEM((K, D), jnp.float32) for _ in range(NB)]
          + [pltpu.SemaphoreType.DMA for _ in range(2 * NB)],
        compiler_params=pltpu.CompilerParams(needs_layout_passes=False),
    )
    def msgpass(h_hbm, src_hbm, dst_hbm, w_hbm, z_hbm, out_hbm,
                acc, src_all, dst_all, w_all, *bufs):
        rows = bufs[:NB]
        gsem = bufs[NB:2 * NB]
        ssem = bufs[2 * NB:3 * NB]
        c = lax.axis_index("c")
        s = lax.axis_index("s")
        wid = s * NC + c
        @pl.when(s < NS - 1)
        def _():
            pltpu.sync_copy(z_hbm.at[pl.ds(0, RPT)],
                            acc.at[pl.ds(s * RPT, RPT)])

        @pl.when(s == NS - 1)
        def _():
            pltpu.sync_copy(z_hbm.at[pl.ds(0, RLAST)],
                            acc.at[pl.ds((NS - 1) * RPT, RLAST)])
        plsc.subcore_barrier()

        def gissue(i, b):
            pltpu.async_copy(h_hbm.at[src_all.at[i]], rows[b], gsem[b])

        def gwait(i, b):
            pltpu.make_async_copy(h_hbm.at[src_all.at[i]], rows[b],
                                  gsem[b]).wait()

        def sissue(i, b):
            pltpu.async_copy(rows[b], acc.at[dst_all.at[i]], ssem[b],
                             add=True)

        def swait(b):
            pltpu.make_async_copy(rows[b], acc.at[dst_all.at[0]],
                                  ssem[b]).wait()

        def scale(i, b):
            @plsc.parallel_loop(0, K, unroll=8)
            def _(e):
                wv = plsc.load_gather(
                    w_all, [jnp.full((LANES,), i * K + e, jnp.int32)])
                for j in range(D // LANES):
                    sl = pl.ds(j * LANES, LANES)
                    rows[b][e, sl] = rows[b][e, sl] * wv

        def stage(p, carry0):
            pltpu.sync_copy(src_hbm.at[wid, p], src_all)
            pltpu.sync_copy(dst_hbm.at[wid, p], dst_all)
            pltpu.sync_copy(w_hbm.at[wid, p], w_all)
            for b in range(NB - 1):
                gissue(b, b)

            def group(g, carry):
                for b in range(NB):
                    i = g * NB + b
                    gwait(i, b)
                    scale(i, b)
                    sissue(i, b)
                    j = i + NB - 1
                    bj = (b + NB - 1) % NB

                    @pl.when(jnp.logical_and(i > 0, j < CP))
                    def _():
                        swait(bj)

                    @pl.when(j < CP)
                    def _():
                        gissue(j, bj)
                return carry

            lax.fori_loop(0, NG, group, 0)
            for t in range(TAIL):
                i = NG * NB + t
                b = i % NB
                gwait(i, b)
                scale(i, b)
                sissue(i, b)
                j = i + NB - 1
                bj = (b + NB - 1) % NB
                if i > 0 and j < CP:
                    swait(bj)
                if j < CP:
                    gissue(j, bj)
            for b in range(NB):
                swait(b)
            return carry0

        lax.fori_loop(0, PH, stage, 0)
        plsc.subcore_barrier()

        @pl.when(s < NS - 1)
        def _():
            pltpu.sync_copy(acc.at[pl.ds(s * RPT, RPT)],
                            out_hbm.at[c, pl.ds(s * RPT, RPT)])

        @pl.when(s == NS - 1)
        def _():
            pltpu.sync_copy(acc.at[pl.ds((NS - 1) * RPT, RLAST)],
                            out_hbm.at[c, pl.ds((NS - 1) * RPT, RLAST)])

    return msgpass


@functools.lru_cache(maxsize=None)
def _make_dense0(N, D, BN):
    def body(x_ref, w_ref, b_ref, o_ref):
        h = jnp.dot(x_ref[...], w_ref[...], preferred_element_type=jnp.float32)
        o_ref[...] = jnp.maximum(h + b_ref[...], 0.0)

    return pl.pallas_call(
        body,
        grid=(N // BN,),
        in_specs=[
            pl.BlockSpec((BN, D), lambda i: (i, 0)),
            pl.BlockSpec((D, D), lambda i: (0, 0)),
            pl.BlockSpec((1, D), lambda i: (0, 0)),
        ],
        out_specs=pl.BlockSpec((BN, D), lambda i: (i, 0)),
        out_shape=jax.ShapeDtypeStruct((N, D), jnp.float32),
    )


@functools.lru_cache(maxsize=None)
def _make_combine(N, D, BN, beta, with_fc, A=None):
    def body(p_ref, x0_ref, w_ref, *rest):
        agg = p_ref[0] + p_ref[1]
        hh = (1.0 - ALPHA) * agg + ALPHA * x0_ref[...]
        hw = jnp.dot(hh, w_ref[...], preferred_element_type=jnp.float32)
        h = jnp.maximum((1.0 - beta) * hh + beta * hw, 0.0)
        if with_fc:
            wfc_ref, bfc_ref, aidx_ref, o_ref, anc_ref = rest
            hfc = (
                jnp.dot(h, wfc_ref[...], preferred_element_type=jnp.float32)
                + bfc_ref[...])
            o_ref[...] = hfc
            i = pl.program_id(0)
            row_ids = (jax.lax.broadcasted_iota(jnp.int32, (BN, A), 0)
                       + i * BN)
            onehot = (row_ids == aidx_ref[...]).astype(jnp.float32)
            part = jax.lax.dot_general(
                onehot, hfc, (((0,), (0,)), ((), ())),
                preferred_element_type=jnp.float32)

            @pl.when(i == 0)
            def _():
                anc_ref[...] = jnp.zeros_like(anc_ref)

            anc_ref[...] += part
        else:
            (o_ref,) = rest
            o_ref[...] = h

    in_specs = [
        pl.BlockSpec((NC, BN, D), lambda i: (0, i, 0)),
        pl.BlockSpec((BN, D), lambda i: (i, 0)),
        pl.BlockSpec((D, D), lambda i: (0, 0)),
    ]
    if with_fc:
        in_specs += [
            pl.BlockSpec((D, D), lambda i: (0, 0)),
            pl.BlockSpec((1, D), lambda i: (0, 0)),
            pl.BlockSpec((1, A), lambda i: (0, 0)),
        ]
        out_specs = [
            pl.BlockSpec((BN, D), lambda i: (i, 0)),
            pl.BlockSpec((A, D), lambda i: (0, 0)),
        ]
        out_shape = [
            jax.ShapeDtypeStruct((N, D), jnp.float32),
            jax.ShapeDtypeStruct((A, D), jnp.float32),
        ]
    else:
        out_specs = pl.BlockSpec((BN, D), lambda i: (i, 0))
        out_shape = jax.ShapeDtypeStruct((N, D), jnp.float32)
    return pl.pallas_call(
        body,
        grid=(N // BN,),
        in_specs=in_specs,
        out_specs=out_specs,
        out_shape=out_shape,
    )


@functools.lru_cache(maxsize=None)
def _make_head(N, D, A, C, BN):
    def body(h_ref, anc_ref, wc_ref, bc_ref, sims_ref, log_ref):
        hn = _l2n(h_ref[...])
        an = _l2n(anc_ref[...])
        s = jnp.dot(hn, an.T, preferred_element_type=jnp.float32)
        sims_ref[...] = s
        out = _l2n(s)
        log_ref[...] = (
            jnp.dot(out, wc_ref[...], preferred_element_type=jnp.float32)
            + bc_ref[...])

    return pl.pallas_call(
        body,
        grid=(N // BN,),
        in_specs=[
            pl.BlockSpec((BN, D), lambda i: (i, 0)),
            pl.BlockSpec((A, D), lambda i: (0, 0)),
            pl.BlockSpec((A, C), lambda i: (0, 0)),
            pl.BlockSpec((1, C), lambda i: (0, 0)),
        ],
        out_specs=[
            pl.BlockSpec((BN, A), lambda i: (i, 0)),
            pl.BlockSpec((BN, C), lambda i: (i, 0)),
        ],
        out_shape=[
            jax.ShapeDtypeStruct((N, A), jnp.float32),
            jax.ShapeDtypeStruct((N, C), jnp.float32),
        ],
    )


def kernel(x, edge_index, edge_weight, anchor_idxs, W_hidden, b_hidden,
           W_conv1, W_conv2, W_fc, b_fc, W_class, b_class):
    N, D = x.shape
    E = edge_weight.shape[0]
    A = anchor_idxs.shape[0]
    C = W_class.shape[1]
    BN = 1000
    K = 40
    PH = 5
    NB = 5
    EW = E // NW
    CH = EW // K
    CP = CH // PH

    msgpass = _make_msgpass(N, D, E, K, PH, NB)
    dense0 = _make_dense0(N, D, BN)
    beta1 = math.log(THETA / 1.0 + 1.0)
    beta2 = math.log(THETA / 2.0 + 1.0)
    comb1 = _make_combine(N, D, BN, beta1, False)
    comb2 = _make_combine(N, D, BN, beta2, True, A)
    head = _make_head(N, D, A, C, BN)

    src_r = edge_index[0].reshape(NW, PH, CP, K)
    dst_r = edge_index[1].reshape(NW, PH, CP, K)
    w_r = edge_weight.reshape(NW, PH, CP * K)
    zeros = jnp.zeros((((N + NS - 1) // NS + 7) // 8 * 8, D), jnp.float32)

    h0 = dense0(x, W_hidden, b_hidden.reshape(1, D))
    p1 = msgpass(h0, src_r, dst_r, w_r, zeros)
    h1 = comb1(p1, h0, W_conv1)
    p2 = msgpass(h1, src_r, dst_r, w_r, zeros)
    hfc, anchors = comb2(p2, h0, W_conv2, W_fc, b_fc.reshape(1, D),
                         anchor_idxs.reshape(1, A))
    sims, logits = head(hfc, anchors, W_class, b_class.reshape(1, C))
    return (logits, sims)


# speedup vs baseline: 9.6957x; 1.0005x over previous
import functools
import math

import jax
import jax.numpy as jnp
from jax import lax
from jax.experimental import pallas as pl
from jax.experimental.pallas import tpu as pltpu
from jax.experimental.pallas import tpu_sc as plsc

ALPHA = 0.1
THETA = 0.5
EPS = 1e-12

NC = 2
NS = 16
NW = NC * NS
LANES = 16


def _l2n(v):
    n = jnp.sqrt(jnp.sum(v * v, axis=-1, keepdims=True))
    return v / jnp.maximum(n, EPS)


@functools.lru_cache(maxsize=None)
def _make_msgpass(N, D, E, K, PH, NB):
    EW = E // NW
    CH = EW // K
    CP = CH // PH
    NG = CP // NB
    TAIL = CP - NG * NB
    assert EW * NW == E and CH * K == EW and CP * PH == CH
    assert K % 8 == 0 and K <= 128
    RPT = ((N + NS - 1) // NS + 7) // 8 * 8
    RLAST = N - RPT * (NS - 1)
    assert RPT % 8 == 0 and 0 < RLAST <= RPT and RLAST % 8 == 0
    mesh = plsc.VectorSubcoreMesh(core_axis_name="c", subcore_axis_name="s")

    @functools.partial(
        pl.kernel,
        out_type=jax.ShapeDtypeStruct((NC, N, D), jnp.float32),
        mesh=mesh,
        scratch_types=[
            pltpu.VMEM_SHARED((N, D), jnp.float32),
            pltpu.VMEM((CP, K), jnp.int32),
            pltpu.VMEM((CP, K), jnp.int32),
            pltpu.VMEM((CP * K,), jnp.float32),
        ] + [pltpu.VMEM((K, D), jnp.float32) for _ in range(NB)]
          + [pltpu.SemaphoreType.DMA for _ in range(2 * NB)],
        compiler_params=pltpu.CompilerParams(needs_layout_passes=False),
    )
    def msgpass(h_hbm, src_hbm, dst_hbm, w_hbm, z_hbm, out_hbm,
                acc, src_all, dst_all, w_all, *bufs):
        rows = bufs[:NB]
        gsem = bufs[NB:2 * NB]
        ssem = bufs[2 * NB:3 * NB]
        c = lax.axis_index("c")
        s = lax.axis_index("s")
        wid = s * NC + c
        @pl.when(s < NS - 1)
        def _():
            pltpu.sync_copy(z_hbm.at[pl.ds(0, RPT)],
                            acc.at[pl.ds(s * RPT, RPT)])

        @pl.when(s == NS - 1)
        def _():
            pltpu.sync_copy(z_hbm.at[pl.ds(0, RLAST)],
                            acc.at[pl.ds((NS - 1) * RPT, RLAST)])
        plsc.subcore_barrier()

        def gissue(i, b):
            pltpu.async_copy(h_hbm.at[src_all.at[i]], rows[b], gsem[b])

        def gwait(i, b):
            pltpu.make_async_copy(h_hbm.at[src_all.at[i]], rows[b],
                                  gsem[b]).wait()

        def sissue(i, b):
            pltpu.async_copy(rows[b], acc.at[dst_all.at[i]], ssem[b],
                             add=True)

        def swait(b):
            pltpu.make_async_copy(rows[b], acc.at[dst_all.at[0]],
                                  ssem[b]).wait()

        def scale(i, b):
            @plsc.parallel_loop(0, K, unroll=8)
            def _(e):
                wv = plsc.load_gather(
                    w_all, [jnp.full((LANES,), i * K + e, jnp.int32)])
                for j in range(D // LANES):
                    sl = pl.ds(j * LANES, LANES)
                    rows[b][e, sl] = rows[b][e, sl] * wv

        def stage(p, carry0):
            pltpu.sync_copy(src_hbm.at[wid, p], src_all)
            pltpu.sync_copy(dst_hbm.at[wid, p], dst_all)
            pltpu.sync_copy(w_hbm.at[wid, p], w_all)
            for b in range(NB - 1):
                gissue(b, b)

            def group(g, carry):
                for b in range(NB):
                    i = g * NB + b
                    gwait(i, b)
                    scale(i, b)
                    sissue(i, b)
                    j = i + NB - 1
                    bj = (b + NB - 1) % NB

                    @pl.when(jnp.logical_and(i > 0, j < CP))
                    def _():
                        swait(bj)

                    @pl.when(j < CP)
                    def _():
                        gissue(j, bj)
                return carry

            lax.fori_loop(0, NG, group, 0)
            for t in range(TAIL):
                i = NG * NB + t
                b = i % NB
                gwait(i, b)
                scale(i, b)
                sissue(i, b)
                j = i + NB - 1
                bj = (b + NB - 1) % NB
                if i > 0 and j < CP:
                    swait(bj)
                if j < CP:
                    gissue(j, bj)
            for b in range(NB):
                swait(b)
            return carry0

        lax.fori_loop(0, PH, stage, 0)
        plsc.subcore_barrier()

        @pl.when(s < NS - 1)
        def _():
            pltpu.sync_copy(acc.at[pl.ds(s * RPT, RPT)],
                            out_hbm.at[c, pl.ds(s * RPT, RPT)])

        @pl.when(s == NS - 1)
        def _():
            pltpu.sync_copy(acc.at[pl.ds((NS - 1) * RPT, RLAST)],
                            out_hbm.at[c, pl.ds((NS - 1) * RPT, RLAST)])

    return msgpass


@functools.lru_cache(maxsize=None)
def _make_dense0(N, D, BN):
    def body(x_ref, w_ref, b_ref, o_ref):
        h = jnp.dot(x_ref[...], w_ref[...], preferred_element_type=jnp.float32)
        o_ref[...] = jnp.maximum(h + b_ref[...], 0.0)

    return pl.pallas_call(
        body,
        grid=(N // BN,),
        in_specs=[
            pl.BlockSpec((BN, D), lambda i: (i, 0)),
            pl.BlockSpec((D, D), lambda i: (0, 0)),
            pl.BlockSpec((1, D), lambda i: (0, 0)),
        ],
        out_specs=pl.BlockSpec((BN, D), lambda i: (i, 0)),
        out_shape=jax.ShapeDtypeStruct((N, D), jnp.float32),
    )


@functools.lru_cache(maxsize=None)
def _make_combine(N, D, BN, beta):
    def body(p_ref, x0_ref, w_ref, o_ref):
        agg = p_ref[0] + p_ref[1]
        hh = (1.0 - ALPHA) * agg + ALPHA * x0_ref[...]
        hw = jnp.dot(hh, w_ref[...], preferred_element_type=jnp.float32)
        o_ref[...] = jnp.maximum((1.0 - beta) * hh + beta * hw, 0.0)

    return pl.pallas_call(
        body,
        grid=(N // BN,),
        in_specs=[
            pl.BlockSpec((NC, BN, D), lambda i: (0, i, 0)),
            pl.BlockSpec((BN, D), lambda i: (i, 0)),
            pl.BlockSpec((D, D), lambda i: (0, 0)),
        ],
        out_specs=pl.BlockSpec((BN, D), lambda i: (i, 0)),
        out_shape=jax.ShapeDtypeStruct((N, D), jnp.float32),
    )


@functools.lru_cache(maxsize=None)
def _make_tail(N, D, A, C, BN, beta):
    G = N // BN

    def body(p_ref, x0_ref, w_ref, wfc_ref, bfc_ref, aidx_ref, wc_ref,
             bc_ref, sims_ref, log_ref, hfc_vmem, anc_vmem):
        i = pl.program_id(0)

        @pl.when(i < G)
        def _():
            agg = p_ref[0] + p_ref[1]
            hh = (1.0 - ALPHA) * agg + ALPHA * x0_ref[...]
            hw = jnp.dot(hh, w_ref[...], preferred_element_type=jnp.float32)
            h = jnp.maximum((1.0 - beta) * hh + beta * hw, 0.0)
            hfc = (
                jnp.dot(h, wfc_ref[...], preferred_element_type=jnp.float32)
                + bfc_ref[...])
            hfc_vmem[pl.ds(i * BN, BN), :] = hfc
            row_ids = (jax.lax.broadcasted_iota(jnp.int32, (BN, A), 0)
                       + i * BN)
            onehot = (row_ids == aidx_ref[...]).astype(jnp.float32)
            part = jax.lax.dot_general(
                onehot, hfc, (((0,), (0,)), ((), ())),
                preferred_element_type=jnp.float32)

            @pl.when(i == 0)
            def _():
                anc_vmem[...] = jnp.zeros_like(anc_vmem)

            anc_vmem[...] += part

        @pl.when(i >= G)
        def _():
            hfc = hfc_vmem[pl.ds((i - G) * BN, BN), :]
            hn = _l2n(hfc)
            an = _l2n(anc_vmem[...])
            s = jnp.dot(hn, an.T, preferred_element_type=jnp.float32)
            sims_ref[...] = s
            out = _l2n(s)
            log_ref[...] = (
                jnp.dot(out, wc_ref[...], preferred_element_type=jnp.float32)
                + bc_ref[...])

    def in_blk(i):
        return jnp.where(i < G, i, G - 1)

    def out_blk(i):
        return jnp.where(i < G, 0, i - G)

    return pl.pallas_call(
        body,
        grid=(2 * G,),
        in_specs=[
            pl.BlockSpec((NC, BN, D), lambda i: (0, in_blk(i), 0)),
            pl.BlockSpec((BN, D), lambda i: (in_blk(i), 0)),
            pl.BlockSpec((D, D), lambda i: (0, 0)),
            pl.BlockSpec((D, D), lambda i: (0, 0)),
            pl.BlockSpec((1, D), lambda i: (0, 0)),
            pl.BlockSpec((1, A), lambda i: (0, 0)),
            pl.BlockSpec((A, C), lambda i: (0, 0)),
            pl.BlockSpec((1, C), lambda i: (0, 0)),
        ],
        out_specs=[
            pl.BlockSpec((BN, A), lambda i: (out_blk(i), 0)),
            pl.BlockSpec((BN, C), lambda i: (out_blk(i), 0)),
        ],
        out_shape=[
            jax.ShapeDtypeStruct((N, A), jnp.float32),
            jax.ShapeDtypeStruct((N, C), jnp.float32),
        ],
        scratch_shapes=[
            pltpu.VMEM((N, D), jnp.float32),
            pltpu.VMEM((A, D), jnp.float32),
        ],
    )


def kernel(x, edge_index, edge_weight, anchor_idxs, W_hidden, b_hidden,
           W_conv1, W_conv2, W_fc, b_fc, W_class, b_class):
    N, D = x.shape
    E = edge_weight.shape[0]
    A = anchor_idxs.shape[0]
    C = W_class.shape[1]
    BN = 1000
    K = 40
    PH = 5
    NB = 5
    EW = E // NW
    CH = EW // K
    CP = CH // PH

    msgpass = _make_msgpass(N, D, E, K, PH, NB)
    dense0 = _make_dense0(N, D, BN)
    beta1 = math.log(THETA / 1.0 + 1.0)
    beta2 = math.log(THETA / 2.0 + 1.0)
    comb1 = _make_combine(N, D, BN, beta1)
    tail = _make_tail(N, D, A, C, BN, beta2)

    src_r = edge_index[0].reshape(NW, PH, CP, K)
    dst_r = edge_index[1].reshape(NW, PH, CP, K)
    w_r = edge_weight.reshape(NW, PH, CP * K)
    zeros = jnp.zeros((((N + NS - 1) // NS + 7) // 8 * 8, D), jnp.float32)

    h0 = dense0(x, W_hidden, b_hidden.reshape(1, D))
    p1 = msgpass(h0, src_r, dst_r, w_r, zeros)
    h1 = comb1(p1, h0, W_conv1)
    p2 = msgpass(h1, src_r, dst_r, w_r, zeros)
    sims, logits = tail(p2, h0, W_conv2, W_fc, b_fc.reshape(1, D),
                        anchor_idxs.reshape(1, A), W_class,
                        b_class.reshape(1, C))
    return (logits, sims)
